# baseline (speedup 1.0000x reference)
import functools

import jax
import jax.numpy as jnp
from jax import lax
from jax.experimental import pallas as pl
from jax.experimental.pallas import tpu as pltpu
from jax.experimental.pallas import tpu_sc as plsc

D = 16
F = 26
FP = F + 1
NNUM = 13
B = 16384
V = 1000000
NW = 32
ROWS_W = B // NW
CHUNK = 128
NCHUNK = ROWS_W // CHUNK
GROWS = CHUNK * FP
IDX_ROWS_W = ROWS_W * FP // 128
IDX_ROWS_PAD = 112

SB = 512
NSB = 999936 // SB
TAIL0 = NSB * SB


def _wid():
    return lax.axis_index("s") * 2 + lax.axis_index("c")


def _tr_body(tv_hbm, tail_hbm, out_hbm, stage0, stage1, stage2,
             obuf0, obuf1, obuf2, in_sem, wb_sem):
    wid = _wid()
    nsb = jnp.where(wid == 0, 62, 61)
    stages = [stage0, stage1, stage2]
    obufs = [obuf0, obuf1, obuf2]
    l16 = lax.iota(jnp.int32, 16) * D

    def issue(k, st):
        s = k * 32 + wid
        c0 = s * SB
        for t in range(4):
            pltpu.async_copy(
                tv_hbm.at[pl.ds(0, 8), pl.ds(c0 + t * 128, 128)],
                st.at[t], in_sem)
            pltpu.async_copy(
                tv_hbm.at[pl.ds(8, 8), pl.ds(c0 + t * 128, 128)],
                st.at[4 + t], in_sem)

    def wait_in(st):
        for t in range(8):
            pltpu.make_async_copy(
                tv_hbm.at[pl.ds(0, 8), pl.ds(0, 128)], st.at[t],
                in_sem).wait()

    def wb_wait(ob):
        pltpu.make_async_copy(
            ob, out_hbm.at[pl.ds(0, SB * D)], wb_sem).wait()

    def process(k, st, ob):
        wait_in(st)
        for t in range(8):
            dd = (t % 4) * 0 + (t // 4) * 8
            for r in range(8):
                d_off = r + dd
                for j0 in range(8):
                    vals = st[t, r, pl.ds(j0 * 16, 16)]
                    cbase = ((t % 4) * 128 + j0 * 16) * D + d_off
                    plsc.store_scatter(ob, [l16 + cbase], vals)
        s = k * 32 + wid
        pltpu.async_copy(ob, out_hbm.at[pl.ds(s * SB * D, SB * D)], wb_sem)

    issue(0, stage0)
    issue(1, stage1)

    def body(kk, carry):
        for bparity in range(3):
            k = kk * 3 + bparity
            st = stages[bparity]
            nxt = stages[(bparity + 2) % 3]
            ob = obufs[bparity]

            @pl.when(k < nsb)
            def _():
                @pl.when(k + 2 < nsb)
                def _():
                    issue(k + 2, nxt)
                @pl.when(k >= 3)
                def _():
                    wb_wait(ob)
                process(k, st, ob)
        return carry

    lax.fori_loop(0, 21, body, 0)
    wb_wait(obuf0)
    wb_wait(obuf1)
    wb_wait(obuf2)

    @pl.when(wid == 1)
    def _():
        pltpu.sync_copy(tail_hbm, obuf0.at[pl.ds(0, 64 * D)])
        pltpu.sync_copy(obuf0.at[pl.ds(0, 64 * D)],
                        out_hbm.at[pl.ds(TAIL0 * D, 64 * D)])


def _sc_body(idx_hbm, nx_hbm, table_hbm, w_hbm, b_hbm, out_hbm,
             idx_v, gbuf0, gbuf1, nx_v, num_buf, w_v, b_v, gsem, osem):
    wid = _wid()
    gbufs = [gbuf0, gbuf1]
    pltpu.sync_copy(w_hbm, w_v)
    pltpu.sync_copy(b_hbm, b_v)
    pltpu.sync_copy(idx_hbm.at[pl.ds(wid * IDX_ROWS_PAD, IDX_ROWS_PAD)],
                    idx_v)

    def fire(c, g):
        handles = []
        for j in range(FP):
            handles.append(pltpu.async_copy(
                table_hbm.at[idx_v.at[c * FP + j]],
                g.at[pl.ds(j * 128, 128)],
                gsem))
        return handles

    def wait_wb(g):
        pltpu.make_async_copy(
            g, out_hbm.at[pl.ds(0, GROWS)], osem).wait()

    handles = fire(0, gbuf0)
    for c in range(NCHUNK):
        g = gbufs[c % 2]
        rb = wid * ROWS_W + c * CHUNK
        pltpu.sync_copy(nx_hbm.at[pl.ds(rb, CHUNK)], nx_v)

        def nbody(i, carry):
            row = nx_v[i, :]
            acc = b_v[:]
            for k in range(NNUM):
                acc = acc + row[k] * w_v[k, :]
            num_buf[i, :] = acc
            return carry

        lax.fori_loop(0, CHUNK, nbody, 0)
        for h in handles:
            h.wait()

        def fbody(i, carry):
            g[i * FP + F, :] = num_buf[i, :]
            return carry

        lax.fori_loop(0, CHUNK, fbody, 0)
        if c + 1 < NCHUNK:
            nxt = gbufs[(c + 1) % 2]
            if c + 1 >= 2:
                wait_wb(nxt)
            handles = fire(c + 1, nxt)
        pltpu.async_copy(g, out_hbm.at[pl.ds(rb * FP, GROWS)], osem)
    wait_wb(gbuf0)
    wait_wb(gbuf1)


def kernel(categorical_x, numerical_x, emb_table, W_num, b_num):
    mesh = plsc.VectorSubcoreMesh(core_axis_name="c", subcore_axis_name="s")

    transpose_run = functools.partial(
        pl.kernel,
        out_type=jax.ShapeDtypeStruct((V * D,), jnp.float32),
        mesh=mesh,
        compiler_params=pltpu.CompilerParams(use_tc_tiling_on_sc=True,
                                             needs_layout_passes=False),
        scratch_types=[
            pltpu.VMEM((8, 8, 128), jnp.float32),
            pltpu.VMEM((8, 8, 128), jnp.float32),
            pltpu.VMEM((8, 8, 128), jnp.float32),
            pltpu.VMEM((SB * D,), jnp.float32),
            pltpu.VMEM((SB * D,), jnp.float32),
            pltpu.VMEM((SB * D,), jnp.float32),
            pltpu.SemaphoreType.DMA,
            pltpu.SemaphoreType.DMA,
        ],
    )(_tr_body)
    tail = emb_table[TAIL0:].reshape(64 * D)
    table_rm = transpose_run(emb_table.T, tail).reshape(V, D)

    idx27 = jnp.concatenate(
        [categorical_x, jnp.zeros((B, 1), jnp.int32)], axis=1)
    idx2d = jnp.pad(idx27.reshape(NW, ROWS_W * FP),
                    ((0, 0), (0, (IDX_ROWS_PAD - IDX_ROWS_W) * 128)))
    idx2d = idx2d.reshape(NW * IDX_ROWS_PAD, 128)
    nxp = jnp.pad(numerical_x, ((0, 0), (0, D - NNUM)))

    gather_run = functools.partial(
        pl.kernel,
        out_type=jax.ShapeDtypeStruct((B * FP, D), jnp.float32),
        mesh=mesh,
        compiler_params=pltpu.CompilerParams(use_tc_tiling_on_sc=False),
        scratch_types=[
            pltpu.VMEM((IDX_ROWS_PAD, 128), jnp.int32),
            pltpu.VMEM((GROWS, D), jnp.float32),
            pltpu.VMEM((GROWS, D), jnp.float32),
            pltpu.VMEM((CHUNK, D), jnp.float32),
            pltpu.VMEM((CHUNK, D), jnp.float32),
            pltpu.VMEM((NNUM, D), jnp.float32),
            pltpu.VMEM((D,), jnp.float32),
            pltpu.SemaphoreType.DMA,
            pltpu.SemaphoreType.DMA,
        ],
    )(_sc_body)
    out = gather_run(idx2d, nxp, table_rm, W_num, b_num)
    return out.reshape(B, FP * D)


# speedup vs baseline: 1.0092x; 1.0092x over previous
import functools

import jax
import jax.numpy as jnp
from jax import lax
from jax.experimental import pallas as pl
from jax.experimental.pallas import tpu as pltpu
from jax.experimental.pallas import tpu_sc as plsc

D = 16
F = 26
FP = F + 1
NNUM = 13
B = 16384
V = 1000000
NW = 32
ROWS_W = B // NW
CHUNK = 128
NCHUNK = ROWS_W // CHUNK
GROWS = CHUNK * FP
IDX_ROWS_W = ROWS_W * FP // 128
IDX_ROWS_PAD = 112

SB = 512
NSB = 999936 // SB
TAIL0 = NSB * SB


def _wid():
    return lax.axis_index("s") * 2 + lax.axis_index("c")


def _tr_body(tv_hbm, tail_hbm, out_hbm, stage0, stage1, stage2,
             obuf0, obuf1, obuf2, in_sem, wb_sem):
    wid = _wid()
    nsb = jnp.where(wid == 0, 62, 61)
    stages = [stage0, stage1, stage2]
    obufs = [obuf0, obuf1, obuf2]
    l16 = lax.iota(jnp.int32, 16) * D

    def issue(k, st):
        s = k * 32 + wid
        c0 = s * SB
        for t in range(4):
            pltpu.async_copy(
                tv_hbm.at[pl.ds(0, 8), pl.ds(c0 + t * 128, 128)],
                st.at[t], in_sem)
            pltpu.async_copy(
                tv_hbm.at[pl.ds(8, 8), pl.ds(c0 + t * 128, 128)],
                st.at[4 + t], in_sem)

    def wait_in(st):
        for t in range(8):
            pltpu.make_async_copy(
                tv_hbm.at[pl.ds(0, 8), pl.ds(0, 128)], st.at[t],
                in_sem).wait()

    def wb_wait(ob):
        pltpu.make_async_copy(
            ob, out_hbm.at[pl.ds(0, SB * D)], wb_sem).wait()

    def process(k, st, ob):
        wait_in(st)
        for t in range(8):
            dd = (t % 4) * 0 + (t // 4) * 8
            for r in range(8):
                d_off = r + dd
                for j0 in range(8):
                    vals = st[t, r, pl.ds(j0 * 16, 16)]
                    cbase = ((t % 4) * 128 + j0 * 16) * D + d_off
                    plsc.store_scatter(ob, [l16 + cbase], vals)
        s = k * 32 + wid
        pltpu.async_copy(ob, out_hbm.at[pl.ds(s * SB * D, SB * D)], wb_sem)

    issue(0, stage0)

    def body(kk, carry):
        for bparity in range(2):
            k = kk * 2 + bparity
            st = stages[bparity]
            other = stages[1 - bparity]
            ob = obufs[bparity]

            @pl.when(k < nsb)
            def _():
                @pl.when(k + 1 < nsb)
                def _():
                    issue(k + 1, other)
                @pl.when(k >= 2)
                def _():
                    wb_wait(ob)
                process(k, st, ob)
        return carry

    lax.fori_loop(0, 31, body, 0)
    wb_wait(obuf0)
    wb_wait(obuf1)

    @pl.when(wid == 1)
    def _():
        pltpu.sync_copy(tail_hbm, obuf0.at[pl.ds(0, 64 * D)])
        pltpu.sync_copy(obuf0.at[pl.ds(0, 64 * D)],
                        out_hbm.at[pl.ds(TAIL0 * D, 64 * D)])


def _sc_body(idx_hbm, nx_hbm, table_hbm, w_hbm, b_hbm, out_hbm,
             idx_v, gbuf0, gbuf1, nx_v, num_buf, w_v, b_v, gsem, osem):
    wid = _wid()
    gbufs = [gbuf0, gbuf1]
    pltpu.sync_copy(w_hbm, w_v)
    pltpu.sync_copy(b_hbm, b_v)
    pltpu.sync_copy(idx_hbm.at[pl.ds(wid * IDX_ROWS_PAD, IDX_ROWS_PAD)],
                    idx_v)

    def fire(c, g):
        handles = []
        for j in range(FP):
            handles.append(pltpu.async_copy(
                table_hbm.at[idx_v.at[c * FP + j]],
                g.at[pl.ds(j * 128, 128)],
                gsem))
        return handles

    def wait_wb(g):
        pltpu.make_async_copy(
            g, out_hbm.at[pl.ds(0, GROWS)], osem).wait()

    handles = fire(0, gbuf0)
    for c in range(NCHUNK):
        g = gbufs[c % 2]
        rb = wid * ROWS_W + c * CHUNK
        pltpu.sync_copy(nx_hbm.at[pl.ds(rb, CHUNK)], nx_v)

        def nbody(i, carry):
            row = nx_v[i, :]
            acc = b_v[:]
            for k in range(NNUM):
                acc = acc + row[k] * w_v[k, :]
            num_buf[i, :] = acc
            return carry

        lax.fori_loop(0, CHUNK, nbody, 0)
        for h in handles:
            h.wait()

        def fbody(i, carry):
            g[i * FP + F, :] = num_buf[i, :]
            return carry

        lax.fori_loop(0, CHUNK, fbody, 0)
        if c + 1 < NCHUNK:
            nxt = gbufs[(c + 1) % 2]
            if c + 1 >= 2:
                wait_wb(nxt)
            handles = fire(c + 1, nxt)
        pltpu.async_copy(g, out_hbm.at[pl.ds(rb * FP, GROWS)], osem)
    wait_wb(gbuf0)
    wait_wb(gbuf1)


def kernel(categorical_x, numerical_x, emb_table, W_num, b_num):
    mesh = plsc.VectorSubcoreMesh(core_axis_name="c", subcore_axis_name="s")

    transpose_run = functools.partial(
        pl.kernel,
        out_type=jax.ShapeDtypeStruct((V * D,), jnp.float32),
        mesh=mesh,
        compiler_params=pltpu.CompilerParams(use_tc_tiling_on_sc=True,
                                             needs_layout_passes=False),
        scratch_types=[
            pltpu.VMEM((8, 8, 128), jnp.float32),
            pltpu.VMEM((8, 8, 128), jnp.float32),
            pltpu.VMEM((8, 8, 128), jnp.float32),
            pltpu.VMEM((SB * D,), jnp.float32),
            pltpu.VMEM((SB * D,), jnp.float32),
            pltpu.VMEM((SB * D,), jnp.float32),
            pltpu.SemaphoreType.DMA,
            pltpu.SemaphoreType.DMA,
        ],
    )(_tr_body)
    tail = emb_table[TAIL0:].reshape(64 * D)
    table_rm = transpose_run(emb_table.T, tail).reshape(V, D)

    idx27 = jnp.concatenate(
        [categorical_x, jnp.zeros((B, 1), jnp.int32)], axis=1)
    idx2d = jnp.pad(idx27.reshape(NW, ROWS_W * FP),
                    ((0, 0), (0, (IDX_ROWS_PAD - IDX_ROWS_W) * 128)))
    idx2d = idx2d.reshape(NW * IDX_ROWS_PAD, 128)
    nxp = jnp.pad(numerical_x, ((0, 0), (0, D - NNUM)))

    gather_run = functools.partial(
        pl.kernel,
        out_type=jax.ShapeDtypeStruct((B * FP, D), jnp.float32),
        mesh=mesh,
        compiler_params=pltpu.CompilerParams(use_tc_tiling_on_sc=False),
        scratch_types=[
            pltpu.VMEM((IDX_ROWS_PAD, 128), jnp.int32),
            pltpu.VMEM((GROWS, D), jnp.float32),
            pltpu.VMEM((GROWS, D), jnp.float32),
            pltpu.VMEM((CHUNK, D), jnp.float32),
            pltpu.VMEM((CHUNK, D), jnp.float32),
            pltpu.VMEM((NNUM, D), jnp.float32),
            pltpu.VMEM((D,), jnp.float32),
            pltpu.SemaphoreType.DMA,
            pltpu.SemaphoreType.DMA,
        ],
    )(_sc_body)
    out = gather_run(idx2d, nxp, table_rm, W_num, b_num)
    return out.reshape(B, FP * D)


# speedup vs baseline: 1.0114x; 1.0022x over previous
import functools

import jax
import jax.numpy as jnp
from jax import lax
from jax.experimental import pallas as pl
from jax.experimental.pallas import tpu as pltpu
from jax.experimental.pallas import tpu_sc as plsc

D = 16
F = 26
FP = F + 1
NNUM = 13
B = 16384
V = 1000000
NW = 32
ROWS_W = B // NW
CHUNK = 128
NCHUNK = ROWS_W // CHUNK
GROWS = CHUNK * FP
IDX_ROWS_W = ROWS_W * FP // 128
IDX_ROWS_PAD = 112

SB = 512
NSB = 999936 // SB
TAIL0 = NSB * SB


def _wid():
    return lax.axis_index("s") * 2 + lax.axis_index("c")


def _tr_body(tv_hbm, tail_hbm, out_hbm, stage0, stage1, stage2,
             obuf0, obuf1, obuf2, in_sem, wb_sem):
    wid = _wid()
    nsb = jnp.where(wid == 0, 62, 61)
    stages = [stage0, stage1, stage2]
    obufs = [obuf0, obuf1, obuf2]
    l16 = lax.iota(jnp.int32, 16) * D

    def issue(k, st):
        s = k * 32 + wid
        c0 = s * SB
        for t in range(4):
            pltpu.async_copy(
                tv_hbm.at[pl.ds(0, 8), pl.ds(c0 + t * 128, 128)],
                st.at[t], in_sem)
            pltpu.async_copy(
                tv_hbm.at[pl.ds(8, 8), pl.ds(c0 + t * 128, 128)],
                st.at[4 + t], in_sem)

    def wait_in(st):
        for t in range(8):
            pltpu.make_async_copy(
                tv_hbm.at[pl.ds(0, 8), pl.ds(0, 128)], st.at[t],
                in_sem).wait()

    def wb_wait(ob):
        pltpu.make_async_copy(
            ob, out_hbm.at[pl.ds(0, SB * D)], wb_sem).wait()

    def process(k, st, ob):
        wait_in(st)
        for t in range(8):
            dd = (t % 4) * 0 + (t // 4) * 8
            for r in range(8):
                d_off = r + dd
                for j0 in range(8):
                    vals = st[t, r, pl.ds(j0 * 16, 16)]
                    cbase = ((t % 4) * 128 + j0 * 16) * D + d_off
                    plsc.store_scatter(ob, [l16 + cbase], vals)
        s = k * 32 + wid
        pltpu.async_copy(ob, out_hbm.at[pl.ds(s * SB * D, SB * D)], wb_sem)

    issue(0, stage0)

    def body(kk, carry):
        for bparity in range(2):
            k = kk * 2 + bparity
            st = stages[bparity]
            other = stages[1 - bparity]
            ob = obufs[bparity]

            @pl.when(k < nsb)
            def _():
                @pl.when(k + 1 < nsb)
                def _():
                    issue(k + 1, other)
                @pl.when(k >= 2)
                def _():
                    wb_wait(ob)
                process(k, st, ob)
        return carry

    lax.fori_loop(0, 31, body, 0)
    wb_wait(obuf0)
    wb_wait(obuf1)

    @pl.when(wid == 1)
    def _():
        pltpu.sync_copy(tail_hbm, obuf0.at[pl.ds(0, 64 * D)])
        pltpu.sync_copy(obuf0.at[pl.ds(0, 64 * D)],
                        out_hbm.at[pl.ds(TAIL0 * D, 64 * D)])


def _sc_body(idx_hbm, nx_hbm, table_hbm, w_hbm, b_hbm, out_hbm,
             idxb0, idxb1, gbuf0, gbuf1, nx_v, num_buf, w_v, b_v,
             gsem, osem):
    wid = _wid()
    gbufs = [gbuf0, gbuf1]
    idxbs = [idxb0, idxb1]
    pltpu.sync_copy(w_hbm, w_v)
    pltpu.sync_copy(b_hbm, b_v)

    def fire(c, g):
        ib = idxbs[c % 2]
        pltpu.sync_copy(
            idx_hbm.at[pl.ds(wid * (IDX_ROWS_PAD * 128) + c * GROWS, GROWS)],
            ib)
        return [pltpu.async_copy(table_hbm.at[ib], g, gsem)]

    def wait_wb(g):
        pltpu.make_async_copy(
            g, out_hbm.at[pl.ds(0, GROWS)], osem).wait()

    handles = fire(0, gbuf0)
    for c in range(NCHUNK):
        g = gbufs[c % 2]
        rb = wid * ROWS_W + c * CHUNK
        pltpu.sync_copy(nx_hbm.at[pl.ds(rb, CHUNK)], nx_v)

        def nbody(i, carry):
            row = nx_v[i, :]
            acc = b_v[:]
            for k in range(NNUM):
                acc = acc + row[k] * w_v[k, :]
            num_buf[i, :] = acc
            return carry

        lax.fori_loop(0, CHUNK, nbody, 0)
        for h in handles:
            h.wait()

        def fbody(i, carry):
            g[i * FP + F, :] = num_buf[i, :]
            return carry

        lax.fori_loop(0, CHUNK, fbody, 0)
        if c + 1 < NCHUNK:
            nxt = gbufs[(c + 1) % 2]
            if c + 1 >= 2:
                wait_wb(nxt)
            handles = fire(c + 1, nxt)
        pltpu.async_copy(g, out_hbm.at[pl.ds(rb * FP, GROWS)], osem)
    wait_wb(gbuf0)
    wait_wb(gbuf1)


def kernel(categorical_x, numerical_x, emb_table, W_num, b_num):
    mesh = plsc.VectorSubcoreMesh(core_axis_name="c", subcore_axis_name="s")

    transpose_run = functools.partial(
        pl.kernel,
        out_type=jax.ShapeDtypeStruct((V * D,), jnp.float32),
        mesh=mesh,
        compiler_params=pltpu.CompilerParams(use_tc_tiling_on_sc=True,
                                             needs_layout_passes=False),
        scratch_types=[
            pltpu.VMEM((8, 8, 128), jnp.float32),
            pltpu.VMEM((8, 8, 128), jnp.float32),
            pltpu.VMEM((8, 8, 128), jnp.float32),
            pltpu.VMEM((SB * D,), jnp.float32),
            pltpu.VMEM((SB * D,), jnp.float32),
            pltpu.VMEM((SB * D,), jnp.float32),
            pltpu.SemaphoreType.DMA,
            pltpu.SemaphoreType.DMA,
        ],
    )(_tr_body)
    tail = emb_table[TAIL0:].reshape(64 * D)
    table_rm = transpose_run(emb_table.T, tail).reshape(V, D)

    idx27 = jnp.concatenate(
        [categorical_x, jnp.zeros((B, 1), jnp.int32)], axis=1)
    idx2d = jnp.pad(idx27.reshape(NW, ROWS_W * FP),
                    ((0, 0), (0, (IDX_ROWS_PAD - IDX_ROWS_W) * 128)))
    idx2d = idx2d.reshape(NW * IDX_ROWS_PAD * 128)
    nxp = jnp.pad(numerical_x, ((0, 0), (0, D - NNUM)))

    gather_run = functools.partial(
        pl.kernel,
        out_type=jax.ShapeDtypeStruct((B * FP, D), jnp.float32),
        mesh=mesh,
        compiler_params=pltpu.CompilerParams(use_tc_tiling_on_sc=False),
        scratch_types=[
            pltpu.VMEM((GROWS,), jnp.int32),
            pltpu.VMEM((GROWS,), jnp.int32),
            pltpu.VMEM((GROWS, D), jnp.float32),
            pltpu.VMEM((GROWS, D), jnp.float32),
            pltpu.VMEM((CHUNK, D), jnp.float32),
            pltpu.VMEM((CHUNK, D), jnp.float32),
            pltpu.VMEM((NNUM, D), jnp.float32),
            pltpu.VMEM((D,), jnp.float32),
            pltpu.SemaphoreType.DMA,
            pltpu.SemaphoreType.DMA,
        ],
    )(_sc_body)
    out = gather_run(idx2d, nxp, table_rm, W_num, b_num)
    return out.reshape(B, FP * D)
